# Initial kernel scaffold; baseline (speedup 1.0000x reference)
import jax
import jax.numpy as jnp
from jax.experimental import pallas as pl


def kernel(x, edge_index, W1, b1, W2, b2, Wa, ba, Wp, bp):
    raise NotImplementedError("write your pallas kernel here")


# speedup vs baseline: 12.5216x; 12.5216x over previous
import functools

import jax
import jax.numpy as jnp
from jax import lax
from jax.experimental import pallas as pl
from jax.experimental.pallas import tpu as pltpu
from jax.experimental.pallas import tpu_sc as plsc

N = 50000
E = 1600000
K = 80
NC = 2
NS = 16
NP = 50048
ZR = NP // NS

_MESH = plsc.VectorSubcoreMesh(
    core_axis_name="c", subcore_axis_name="s", num_cores=NC, num_subcores=NS)

F32 = jnp.float32


def _deg_body(col1d, ones_hbm, zeros8, out, acc, idxc, ones_v, sem):
    c = lax.axis_index("c")
    s = lax.axis_index("s")
    pltpu.sync_copy(zeros8, acc.at[pl.ds(s * ZR, ZR)])
    pltpu.sync_copy(ones_hbm, ones_v)
    plsc.subcore_barrier()
    wid = c * NS + s
    tile_edges = E // (NC * NS)
    base = wid * tile_edges

    def step(t, _):
        e0 = base + t * K
        pltpu.sync_copy(col1d.at[pl.ds(e0, K)], idxc)
        pltpu.sync_copy(ones_v, acc.at[idxc], add=True)
        return 0

    lax.fori_loop(0, tile_edges // K, step, 0)
    plsc.subcore_barrier()
    pltpu.sync_copy(acc.at[pl.ds(s * ZR, ZR)], out.at[c, pl.ds(s * ZR, ZR)])


_deg_kernel = functools.partial(
    pl.kernel,
    out_type=jax.ShapeDtypeStruct((NC, NP, 8), F32),
    mesh=_MESH,
    compiler_params=pltpu.CompilerParams(use_tc_tiling_on_sc=False),
    scratch_types=[
        pltpu.VMEM_SHARED((NP, 8), F32),
        pltpu.VMEM((K,), jnp.int32),
        pltpu.VMEM((K, 8), F32),
        pltpu.SemaphoreType.DMA,
    ],
)(_deg_body)


def _agg1_body(row1d, col1d, table, zeros8, out, acc, idxr, idxc, rows, sem):
    c = lax.axis_index("c")
    s = lax.axis_index("s")
    pltpu.sync_copy(zeros8, acc.at[pl.ds(s * ZR, ZR)])
    plsc.subcore_barrier()
    wid = c * NS + s
    tile_edges = E // (NC * NS)
    base = wid * tile_edges

    def step(t, _):
        e0 = base + t * K
        pltpu.sync_copy(row1d.at[pl.ds(e0, K)], idxr)
        pltpu.sync_copy(col1d.at[pl.ds(e0, K)], idxc)
        pltpu.async_copy(table.at[idxr], rows, sem).wait()
        pltpu.sync_copy(rows, acc.at[idxc], add=True)
        return 0

    lax.fori_loop(0, tile_edges // K, step, 0)
    plsc.subcore_barrier()
    pltpu.sync_copy(acc.at[pl.ds(s * ZR, ZR)], out.at[c, pl.ds(s * ZR, ZR)])


_agg1_kernel = functools.partial(
    pl.kernel,
    out_type=jax.ShapeDtypeStruct((NC, NP, 8), F32),
    mesh=_MESH,
    compiler_params=pltpu.CompilerParams(use_tc_tiling_on_sc=False),
    scratch_types=[
        pltpu.VMEM_SHARED((NP, 8), F32),
        pltpu.VMEM((K,), jnp.int32),
        pltpu.VMEM((K,), jnp.int32),
        pltpu.VMEM((K, 8), F32),
        pltpu.SemaphoreType.DMA,
    ],
)(_agg1_body)


def _agg2_body(row1d, col1d, ta, tb, zeros32, outa, outb,
               acc, idxr, idxc, rows, sem):
    c = lax.axis_index("c")
    s = lax.axis_index("s")
    pltpu.sync_copy(zeros32, acc.at[pl.ds(s * ZR, ZR)])
    plsc.subcore_barrier()
    tile_edges = E // NS
    base = s * tile_edges

    def run(table):
        def step(t, _):
            e0 = base + t * K
            pltpu.sync_copy(row1d.at[pl.ds(e0, K)], idxr)
            pltpu.sync_copy(col1d.at[pl.ds(e0, K)], idxc)
            pltpu.async_copy(table.at[idxr], rows, sem).wait()
            pltpu.sync_copy(rows, acc.at[idxc], add=True)
            return 0

        lax.fori_loop(0, tile_edges // K, step, 0)

    @pl.when(c == 0)
    def _():
        run(ta)

    @pl.when(c == 1)
    def _():
        run(tb)

    plsc.subcore_barrier()

    @pl.when(c == 0)
    def _():
        pltpu.sync_copy(acc.at[pl.ds(s * ZR, ZR)], outa.at[pl.ds(s * ZR, ZR)])

    @pl.when(c == 1)
    def _():
        pltpu.sync_copy(acc.at[pl.ds(s * ZR, ZR)], outb.at[pl.ds(s * ZR, ZR)])


_agg2_kernel = functools.partial(
    pl.kernel,
    out_type=(jax.ShapeDtypeStruct((NP, 32), F32),
              jax.ShapeDtypeStruct((NP, 32), F32)),
    mesh=_MESH,
    compiler_params=pltpu.CompilerParams(use_tc_tiling_on_sc=False),
    scratch_types=[
        pltpu.VMEM_SHARED((NP, 32), F32),
        pltpu.VMEM((K,), jnp.int32),
        pltpu.VMEM((K,), jnp.int32),
        pltpu.VMEM((K, 32), F32),
        pltpu.SemaphoreType.DMA,
    ],
)(_agg2_body)


R = 2000
G = N // R


def _tc1_body(degp_ref, x_ref, dinv_ref, u1p_ref):
    deg = degp_ref[0, :, 0:1] + degp_ref[1, :, 0:1] + 1.0
    dinv = lax.rsqrt(deg)
    dinv_ref[...] = dinv
    u1 = x_ref[...] * dinv
    u1p_ref[...] = jnp.concatenate([u1, jnp.zeros((R, 6), F32)], axis=1)


def _tc1(degp, x):
    return pl.pallas_call(
        _tc1_body,
        grid=(G,),
        in_specs=[
            pl.BlockSpec((NC, R, 8), lambda i: (0, i, 0)),
            pl.BlockSpec((R, 2), lambda i: (i, 0)),
        ],
        out_specs=[
            pl.BlockSpec((R, 1), lambda i: (i, 0)),
            pl.BlockSpec((R, 8), lambda i: (i, 0)),
        ],
        out_shape=[
            jax.ShapeDtypeStruct((N, 1), F32),
            jax.ShapeDtypeStruct((N, 8), F32),
        ],
    )(degp, x)


def _tc2_body(aggp_ref, u1p_ref, dinv_ref, w1_ref, b1_ref, w2_ref,
              u2a_ref, u2b_ref):
    agg = (aggp_ref[0] + aggp_ref[1] + u1p_ref[...])[:, 0:2] * dinv_ref[...]
    h1 = jax.nn.relu(agg[:, 0:1] * w1_ref[0:1, :]
                     + agg[:, 1:2] * w1_ref[1:2, :] + b1_ref[...])
    t = jnp.dot(h1, w2_ref[...], preferred_element_type=F32)
    u2 = t * dinv_ref[...]
    u2a_ref[...] = u2[:, 0:32]
    u2b_ref[...] = u2[:, 32:64]


def _tc2(aggp, u1p, dinv, W1, b1, W2):
    return pl.pallas_call(
        _tc2_body,
        grid=(G,),
        in_specs=[
            pl.BlockSpec((NC, R, 8), lambda i: (0, i, 0)),
            pl.BlockSpec((R, 8), lambda i: (i, 0)),
            pl.BlockSpec((R, 1), lambda i: (i, 0)),
            pl.BlockSpec((2, 128), lambda i: (0, 0)),
            pl.BlockSpec((1, 128), lambda i: (0, 0)),
            pl.BlockSpec((128, 64), lambda i: (0, 0)),
        ],
        out_specs=[
            pl.BlockSpec((R, 32), lambda i: (i, 0)),
            pl.BlockSpec((R, 32), lambda i: (i, 0)),
        ],
        out_shape=[
            jax.ShapeDtypeStruct((N, 32), F32),
            jax.ShapeDtypeStruct((N, 32), F32),
        ],
    )(aggp, u1p, dinv, W1, b1, W2)


def _tc3_body(a2a_ref, a2b_ref, u2a_ref, u2b_ref, dinv_ref, b2_ref,
              wa_ref, ba_ref, wp_ref, bp_ref, out_ref):
    dinv = dinv_ref[...]
    h2a = jax.nn.relu((a2a_ref[...] + u2a_ref[...]) * dinv + b2_ref[:, 0:32])
    h2b = jax.nn.relu((a2b_ref[...] + u2b_ref[...]) * dinv + b2_ref[:, 32:64])
    alog = (jnp.sum(h2a * wa_ref[:, 0:32], axis=1, keepdims=True)
            + jnp.sum(h2b * wa_ref[:, 32:64], axis=1, keepdims=True)
            + ba_ref[0, 0])
    attn = jax.nn.sigmoid(alog)
    plog = (jnp.sum(h2a * wp_ref[:, 0:32], axis=1, keepdims=True)
            + jnp.sum(h2b * wp_ref[:, 32:64], axis=1, keepdims=True))
    out_ref[...] = jax.nn.sigmoid(attn * plog + bp_ref[0, 0])


def _tc3(a2a, a2b, u2a, u2b, dinv, b2, wa, ba, wp, bp):
    return pl.pallas_call(
        _tc3_body,
        grid=(G,),
        in_specs=[
            pl.BlockSpec((R, 32), lambda i: (i, 0)),
            pl.BlockSpec((R, 32), lambda i: (i, 0)),
            pl.BlockSpec((R, 32), lambda i: (i, 0)),
            pl.BlockSpec((R, 32), lambda i: (i, 0)),
            pl.BlockSpec((R, 1), lambda i: (i, 0)),
            pl.BlockSpec((1, 64), lambda i: (0, 0)),
            pl.BlockSpec((1, 64), lambda i: (0, 0)),
            pl.BlockSpec((1, 1), lambda i: (0, 0)),
            pl.BlockSpec((1, 64), lambda i: (0, 0)),
            pl.BlockSpec((1, 1), lambda i: (0, 0)),
        ],
        out_specs=pl.BlockSpec((R, 1), lambda i: (i, 0)),
        out_shape=jax.ShapeDtypeStruct((N, 1), F32),
    )(a2a, a2b, u2a, u2b, dinv, b2, wa, ba, wp, bp)


def kernel(x, edge_index, W1, b1, W2, b2, Wa, ba, Wp, bp):
    row1d = edge_index[0]
    col1d = edge_index[1]
    ones8 = jnp.ones((K, 8), F32)
    zeros8 = jnp.zeros((ZR, 8), F32)
    zeros32 = jnp.zeros((ZR, 32), F32)

    degp = _deg_kernel(col1d, ones8, zeros8)
    dinv, u1p = _tc1(degp, x)
    aggp = _agg1_kernel(row1d, col1d, u1p, zeros8)
    u2a, u2b = _tc2(aggp, u1p, dinv, W1, b1.reshape(1, 128), W2)
    a2a, a2b = _agg2_kernel(row1d, col1d, u2a, u2b, zeros32)
    out = _tc3(a2a[:N], a2b[:N], u2a, u2b, dinv, b2.reshape(1, 64),
               Wa.reshape(1, 64), ba.reshape(1, 1),
               Wp.reshape(1, 64), bp.reshape(1, 1))
    return out.reshape(N)


# speedup vs baseline: 33.2541x; 2.6557x over previous
import functools

import jax
import jax.numpy as jnp
from jax import lax
from jax.experimental import pallas as pl
from jax.experimental.pallas import tpu as pltpu
from jax.experimental.pallas import tpu_sc as plsc

N = 50000
E = 1600000
K = 128
NB = 5
NC = 2
NS = 16
NP = 50048
ZR = NP // NS
CH = 400
EP = NC * NS * CH * K
CH2 = 2 * CH

_MESH = plsc.VectorSubcoreMesh(
    core_axis_name="c", subcore_axis_name="s", num_cores=NC, num_subcores=NS)

F32 = jnp.float32


def _fire_idx(src1d, dst, pp, o, base, sem):
    for b in range(NB):
        e0 = base + (o * NB + b) * K
        pltpu.async_copy(src1d.at[pl.ds(e0, K)], dst.at[pp, b], sem.at[pp])


def _drain_idx(src1d, dst, pp, sem, n):
    for _ in range(n):
        pltpu.make_async_copy(src1d.at[pl.ds(0, K)], dst.at[pp, 0],
                              sem.at[pp]).wait()


def _agg_pipeline(src_r, src_c, table, acc, idxr, idxc, rows,
                  semi, semg, sems, base, nblocks):
    def fire_scatters(pp):
        for b in range(NB):
            pltpu.async_copy(rows.at[pp, b], acc.at[idxc.at[pp, b]],
                             sems.at[pp], add=True)

    def drain_scatters(pp):
        for _ in range(NB):
            pltpu.make_async_copy(rows.at[pp, 0], acc.at[pl.ds(0, K)],
                                  sems.at[pp]).wait()

    zero = jnp.int32(0)
    _fire_idx(src_r, idxr, 0, zero, base, semi)
    _fire_idx(src_c, idxc, 0, zero, base, semi)
    _fire_idx(src_r, idxr, 1, zero + 1, base, semi)

    def body(oo, _):
        for p in (0, 1):
            q = 1 - p
            o = 2 * oo + p
            _drain_idx(src_r, idxr, p, semi, 2 * NB)
            g = [pltpu.async_copy(table.at[idxr.at[p, b]], rows.at[p, b],
                                  semg.at[p]) for b in range(NB)]
            for d in g:
                d.wait()
            _fire_idx(src_r, idxr, p, jnp.minimum(o + 2, nblocks - 1),
                      base, semi)
            if p == 0:
                @pl.when(oo > 0)
                def _():
                    drain_scatters(q)
            else:
                drain_scatters(q)
            _fire_idx(src_c, idxc, q, jnp.minimum(o + 1, nblocks - 1),
                      base, semi)
            fire_scatters(p)
        return 0

    lax.fori_loop(0, nblocks // 2, body, 0)
    _drain_idx(src_r, idxr, 0, semi, 2 * NB)
    _drain_idx(src_r, idxr, 1, semi, NB)
    drain_scatters(1)


def _deg_body(col1d, ones_hbm, zeros8, out, acc, idxc, ones_v, semi, sems):
    c = lax.axis_index("c")
    s = lax.axis_index("s")
    pltpu.sync_copy(zeros8, acc.at[pl.ds(s * ZR, ZR)])
    pltpu.sync_copy(ones_hbm, ones_v)
    plsc.subcore_barrier()
    wid = c * NS + s
    base = wid * CH * K
    nblocks = CH // NB

    def fire_scatters(pp):
        for b in range(NB):
            pltpu.async_copy(ones_v, acc.at[idxc.at[pp, b]],
                             sems.at[pp], add=True)

    def drain_scatters(pp):
        for _ in range(NB):
            pltpu.make_async_copy(ones_v, acc.at[pl.ds(0, K)],
                                  sems.at[pp]).wait()

    _fire_idx(col1d, idxc, 0, jnp.int32(0), base, semi)

    def body(oo, _):
        for p in (0, 1):
            q = 1 - p
            o = 2 * oo + p
            _drain_idx(col1d, idxc, p, semi, NB)
            if p == 0:
                @pl.when(oo > 0)
                def _():
                    drain_scatters(q)
            else:
                drain_scatters(q)
            _fire_idx(col1d, idxc, q, jnp.minimum(o + 1, nblocks - 1),
                      base, semi)
            fire_scatters(p)
        return 0

    lax.fori_loop(0, nblocks // 2, body, 0)
    _drain_idx(col1d, idxc, 0, semi, NB)
    drain_scatters(1)

    plsc.subcore_barrier()
    pltpu.sync_copy(acc.at[pl.ds(s * ZR, ZR)], out.at[c, pl.ds(s * ZR, ZR)])


_deg_kernel = functools.partial(
    pl.kernel,
    out_type=jax.ShapeDtypeStruct((NC, NP, 8), F32),
    mesh=_MESH,
    compiler_params=pltpu.CompilerParams(use_tc_tiling_on_sc=False),
    scratch_types=[
        pltpu.VMEM_SHARED((NP, 8), F32),
        pltpu.VMEM((2, NB, K), jnp.int32),
        pltpu.VMEM((K, 8), F32),
        pltpu.SemaphoreType.DMA((2,)),
        pltpu.SemaphoreType.DMA((2,)),
    ],
)(_deg_body)


def _agg1_body(row1d, col1d, table, zeros8, out,
               acc, idxr, idxc, rows, semi, semg, sems):
    c = lax.axis_index("c")
    s = lax.axis_index("s")
    pltpu.sync_copy(zeros8, acc.at[pl.ds(s * ZR, ZR)])
    plsc.subcore_barrier()
    wid = c * NS + s
    base = wid * CH * K
    _agg_pipeline(row1d, col1d, table, acc, idxr, idxc, rows,
                  semi, semg, sems, base, CH // NB)
    plsc.subcore_barrier()
    pltpu.sync_copy(acc.at[pl.ds(s * ZR, ZR)], out.at[c, pl.ds(s * ZR, ZR)])


_agg1_kernel = functools.partial(
    pl.kernel,
    out_type=jax.ShapeDtypeStruct((NC, NP, 8), F32),
    mesh=_MESH,
    compiler_params=pltpu.CompilerParams(use_tc_tiling_on_sc=False),
    scratch_types=[
        pltpu.VMEM_SHARED((NP, 8), F32),
        pltpu.VMEM((2, NB, K), jnp.int32),
        pltpu.VMEM((2, NB, K), jnp.int32),
        pltpu.VMEM((2, NB, K, 8), F32),
        pltpu.SemaphoreType.DMA((2,)),
        pltpu.SemaphoreType.DMA((2,)),
        pltpu.SemaphoreType.DMA((2,)),
    ],
)(_agg1_body)


def _agg2_body(row1d, col1d, t0, t1, t2, t3, zeros16,
               out0, out1, out2, out3,
               acc, idxr, idxc, rows, semi, semg, sems):
    c = lax.axis_index("c")
    s = lax.axis_index("s")
    base = s * CH2 * K
    tabs = (t0, t1, t2, t3)
    outs = (out0, out1, out2, out3)
    for q in (0, 1):
        pltpu.sync_copy(zeros16, acc.at[pl.ds(s * ZR, ZR)])
        plsc.subcore_barrier()

        @pl.when(c == 0)
        def _():
            _agg_pipeline(row1d, col1d, tabs[q], acc, idxr, idxc, rows,
                          semi, semg, sems, base, CH2 // NB)

        @pl.when(c == 1)
        def _():
            _agg_pipeline(row1d, col1d, tabs[2 + q], acc, idxr, idxc, rows,
                          semi, semg, sems, base, CH2 // NB)

        plsc.subcore_barrier()

        @pl.when(c == 0)
        def _():
            pltpu.sync_copy(acc.at[pl.ds(s * ZR, ZR)],
                            outs[q].at[pl.ds(s * ZR, ZR)])

        @pl.when(c == 1)
        def _():
            pltpu.sync_copy(acc.at[pl.ds(s * ZR, ZR)],
                            outs[2 + q].at[pl.ds(s * ZR, ZR)])
        plsc.subcore_barrier()


_agg2_kernel = functools.partial(
    pl.kernel,
    out_type=tuple(jax.ShapeDtypeStruct((NP, 16), F32) for _ in range(4)),
    mesh=_MESH,
    compiler_params=pltpu.CompilerParams(use_tc_tiling_on_sc=False),
    scratch_types=[
        pltpu.VMEM_SHARED((NP, 16), F32),
        pltpu.VMEM((2, NB, K), jnp.int32),
        pltpu.VMEM((2, NB, K), jnp.int32),
        pltpu.VMEM((2, NB, K, 16), F32),
        pltpu.SemaphoreType.DMA((2,)),
        pltpu.SemaphoreType.DMA((2,)),
        pltpu.SemaphoreType.DMA((2,)),
    ],
)(_agg2_body)


R = 2000
G = N // R


def _tc1_body(degp_ref, x_ref, dinv_ref, u1p_ref):
    deg = degp_ref[0, :, 0:1] + degp_ref[1, :, 0:1] + 1.0
    dinv = lax.rsqrt(deg)
    dinv_ref[...] = dinv
    u1 = x_ref[...] * dinv
    u1p_ref[...] = jnp.concatenate([u1, jnp.zeros((R, 6), F32)], axis=1)


def _tc1(degp, x):
    return pl.pallas_call(
        _tc1_body,
        grid=(G,),
        in_specs=[
            pl.BlockSpec((NC, R, 8), lambda i: (0, i, 0)),
            pl.BlockSpec((R, 2), lambda i: (i, 0)),
        ],
        out_specs=[
            pl.BlockSpec((R, 1), lambda i: (i, 0)),
            pl.BlockSpec((R, 8), lambda i: (i, 0)),
        ],
        out_shape=[
            jax.ShapeDtypeStruct((N, 1), F32),
            jax.ShapeDtypeStruct((N, 8), F32),
        ],
    )(degp, x)


def _tc2_body(aggp_ref, u1p_ref, dinv_ref, w1_ref, b1_ref, w2_ref,
              u2q0_ref, u2q1_ref, u2q2_ref, u2q3_ref):
    agg = (aggp_ref[0] + aggp_ref[1] + u1p_ref[...])[:, 0:2] * dinv_ref[...]
    h1 = jax.nn.relu(agg[:, 0:1] * w1_ref[0:1, :]
                     + agg[:, 1:2] * w1_ref[1:2, :] + b1_ref[...])
    t = jnp.dot(h1, w2_ref[...], preferred_element_type=F32)
    u2 = t * dinv_ref[...]
    u2q0_ref[...] = u2[:, 0:16]
    u2q1_ref[...] = u2[:, 16:32]
    u2q2_ref[...] = u2[:, 32:48]
    u2q3_ref[...] = u2[:, 48:64]


def _tc2(aggp, u1p, dinv, W1, b1, W2):
    return pl.pallas_call(
        _tc2_body,
        grid=(G,),
        in_specs=[
            pl.BlockSpec((NC, R, 8), lambda i: (0, i, 0)),
            pl.BlockSpec((R, 8), lambda i: (i, 0)),
            pl.BlockSpec((R, 1), lambda i: (i, 0)),
            pl.BlockSpec((2, 128), lambda i: (0, 0)),
            pl.BlockSpec((1, 128), lambda i: (0, 0)),
            pl.BlockSpec((128, 64), lambda i: (0, 0)),
        ],
        out_specs=[pl.BlockSpec((R, 16), lambda i: (i, 0))
                   for _ in range(4)],
        out_shape=[jax.ShapeDtypeStruct((N, 16), F32) for _ in range(4)],
    )(aggp, u1p, dinv, W1, b1, W2)


def _tc3_body(a0_ref, a1_ref, a2_ref, a3_ref,
              u0_ref, u1_ref, u2_ref, u3_ref, dinv_ref, b2_ref,
              wa_ref, ba_ref, wp_ref, bp_ref, out_ref):
    dinv = dinv_ref[...]
    arefs = (a0_ref, a1_ref, a2_ref, a3_ref)
    urefs = (u0_ref, u1_ref, u2_ref, u3_ref)
    alog = ba_ref[0, 0]
    plog = 0.0
    for qq in range(4):
        lo = 16 * qq
        h2q = jax.nn.relu((arefs[qq][...] + urefs[qq][...]) * dinv
                          + b2_ref[:, lo:lo + 16])
        alog = alog + jnp.sum(h2q * wa_ref[:, lo:lo + 16], axis=1,
                              keepdims=True)
        plog = plog + jnp.sum(h2q * wp_ref[:, lo:lo + 16], axis=1,
                              keepdims=True)
    attn = jax.nn.sigmoid(alog)
    out_ref[...] = jax.nn.sigmoid(attn * plog + bp_ref[0, 0])


def _tc3(aq, uq, dinv, b2, wa, ba, wp, bp):
    return pl.pallas_call(
        _tc3_body,
        grid=(G,),
        in_specs=[pl.BlockSpec((R, 16), lambda i: (i, 0))
                  for _ in range(8)] + [
            pl.BlockSpec((R, 1), lambda i: (i, 0)),
            pl.BlockSpec((1, 64), lambda i: (0, 0)),
            pl.BlockSpec((1, 64), lambda i: (0, 0)),
            pl.BlockSpec((1, 1), lambda i: (0, 0)),
            pl.BlockSpec((1, 64), lambda i: (0, 0)),
            pl.BlockSpec((1, 1), lambda i: (0, 0)),
        ],
        out_specs=pl.BlockSpec((R, 1), lambda i: (i, 0)),
        out_shape=jax.ShapeDtypeStruct((N, 1), F32),
    )(*aq, *uq, dinv, b2, wa, ba, wp, bp)


def kernel(x, edge_index, W1, b1, W2, b2, Wa, ba, Wp, bp):
    pad = EP - E
    pad_rows = (jnp.arange(pad, dtype=jnp.int32) % N)
    pad_cols = N + (jnp.arange(pad, dtype=jnp.int32) % (NP - N))
    row1d = jnp.concatenate([edge_index[0], pad_rows])
    col1d = jnp.concatenate([edge_index[1], pad_cols])
    ones8 = jnp.ones((K, 8), F32)
    zeros8 = jnp.zeros((ZR, 8), F32)
    zeros16 = jnp.zeros((ZR, 16), F32)

    degp = _deg_kernel(col1d, ones8, zeros8)
    dinv, u1p = _tc1(degp, x)
    aggp = _agg1_kernel(row1d, col1d, u1p, zeros8)
    u2q = _tc2(aggp, u1p, dinv, W1, b1.reshape(1, 128), W2)
    a2q = _agg2_kernel(row1d, col1d, *u2q, zeros16)
    out = _tc3([a[:N] for a in a2q], u2q, dinv, b2.reshape(1, 64),
               Wa.reshape(1, 64), ba.reshape(1, 1),
               Wp.reshape(1, 64), bp.reshape(1, 1))
    return out.reshape(N)


# speedup vs baseline: 37.9694x; 1.1418x over previous
import functools

import jax
import jax.numpy as jnp
from jax import lax
from jax.experimental import pallas as pl
from jax.experimental.pallas import tpu as pltpu
from jax.experimental.pallas import tpu_sc as plsc

N = 50000
E = 1600000
K = 128
NB = 8
NC = 2
NS = 16
NP = 50048
ZR = NP // NS
CH = 400
EP = NC * NS * CH * K
CH2 = 2 * CH

_MESH = plsc.VectorSubcoreMesh(
    core_axis_name="c", subcore_axis_name="s", num_cores=NC, num_subcores=NS)

F32 = jnp.float32


def _fire_idx(src3, dst, pp, o, base_row, sem):
    pltpu.async_copy(src3.at[pl.ds(base_row + o * NB, NB)], dst.at[pp],
                     sem.at[pp])


def _drain_idx(src3, dst, pp, sem, n):
    for _ in range(n):
        pltpu.make_async_copy(src3.at[pl.ds(0, NB)], dst.at[pp],
                              sem.at[pp]).wait()


def _agg_pipeline(src_r, src_c, table, acc, idxr, idxc, rows,
                  semi, semg, sems, base, nblocks):
    def fire_scatters(pp):
        for b in range(NB):
            pltpu.async_copy(rows.at[pp, b], acc.at[idxc.at[pp, b, 0]],
                             sems.at[pp], add=True)

    def drain_scatters(pp):
        for _ in range(NB):
            pltpu.make_async_copy(rows.at[pp, 0], acc.at[pl.ds(0, K)],
                                  sems.at[pp]).wait()

    zero = jnp.int32(0)
    _fire_idx(src_r, idxr, 0, zero, base, semi)
    _fire_idx(src_c, idxc, 0, zero, base, semi)
    _fire_idx(src_r, idxr, 1, zero + 1, base, semi)

    def body(oo, _):
        for p in (0, 1):
            q = 1 - p
            o = 2 * oo + p
            _drain_idx(src_r, idxr, p, semi, 2)
            g = [pltpu.async_copy(table.at[idxr.at[p, b, 0]], rows.at[p, b],
                                  semg.at[p]) for b in range(NB)]
            for d in g:
                d.wait()
            _fire_idx(src_r, idxr, p, jnp.minimum(o + 2, nblocks - 1),
                      base, semi)
            if p == 0:
                @pl.when(oo > 0)
                def _():
                    drain_scatters(q)
            else:
                drain_scatters(q)
            _fire_idx(src_c, idxc, q, jnp.minimum(o + 1, nblocks - 1),
                      base, semi)
            fire_scatters(p)
        return 0

    lax.fori_loop(0, nblocks // 2, body, 0)
    _drain_idx(src_r, idxr, 0, semi, 2)
    _drain_idx(src_r, idxr, 1, semi, 1)
    drain_scatters(1)


def _deg_body(col1d, ones_hbm, zeros8, out, acc, idxc, ones_v, semi, sems):
    c = lax.axis_index("c")
    s = lax.axis_index("s")
    pltpu.sync_copy(zeros8, acc.at[pl.ds(s * ZR, ZR)])
    pltpu.sync_copy(ones_hbm, ones_v)
    plsc.subcore_barrier()
    wid = c * NS + s
    base = wid * CH
    nblocks = CH // NB

    def fire_scatters(pp):
        for b in range(NB):
            pltpu.async_copy(ones_v, acc.at[idxc.at[pp, b, 0]],
                             sems.at[pp], add=True)

    def drain_scatters(pp):
        for _ in range(NB):
            pltpu.make_async_copy(ones_v, acc.at[pl.ds(0, K)],
                                  sems.at[pp]).wait()

    _fire_idx(col1d, idxc, 0, jnp.int32(0), base, semi)

    def body(oo, _):
        for p in (0, 1):
            q = 1 - p
            o = 2 * oo + p
            _drain_idx(col1d, idxc, p, semi, 1)
            if p == 0:
                @pl.when(oo > 0)
                def _():
                    drain_scatters(q)
            else:
                drain_scatters(q)
            _fire_idx(col1d, idxc, q, jnp.minimum(o + 1, nblocks - 1),
                      base, semi)
            fire_scatters(p)
        return 0

    lax.fori_loop(0, nblocks // 2, body, 0)
    _drain_idx(col1d, idxc, 0, semi, 1)
    drain_scatters(1)

    plsc.subcore_barrier()
    pltpu.sync_copy(acc.at[pl.ds(s * ZR, ZR)], out.at[c, pl.ds(s * ZR, ZR)])


_deg_kernel = functools.partial(
    pl.kernel,
    out_type=jax.ShapeDtypeStruct((NC, NP, 8), F32),
    mesh=_MESH,
    compiler_params=pltpu.CompilerParams(use_tc_tiling_on_sc=False),
    scratch_types=[
        pltpu.VMEM_SHARED((NP, 8), F32),
        pltpu.VMEM((2, NB, 1, K), jnp.int32),
        pltpu.VMEM((K, 8), F32),
        pltpu.SemaphoreType.DMA((2,)),
        pltpu.SemaphoreType.DMA((2,)),
    ],
)(_deg_body)


def _agg1_body(row1d, col1d, table, zeros8, out,
               acc, idxr, idxc, rows, semi, semg, sems):
    c = lax.axis_index("c")
    s = lax.axis_index("s")
    pltpu.sync_copy(zeros8, acc.at[pl.ds(s * ZR, ZR)])
    plsc.subcore_barrier()
    wid = c * NS + s
    base = wid * CH
    _agg_pipeline(row1d, col1d, table, acc, idxr, idxc, rows,
                  semi, semg, sems, base, CH // NB)
    plsc.subcore_barrier()
    pltpu.sync_copy(acc.at[pl.ds(s * ZR, ZR)], out.at[c, pl.ds(s * ZR, ZR)])


_agg1_kernel = functools.partial(
    pl.kernel,
    out_type=jax.ShapeDtypeStruct((NC, NP, 8), F32),
    mesh=_MESH,
    compiler_params=pltpu.CompilerParams(use_tc_tiling_on_sc=False),
    scratch_types=[
        pltpu.VMEM_SHARED((NP, 8), F32),
        pltpu.VMEM((2, NB, 1, K), jnp.int32),
        pltpu.VMEM((2, NB, 1, K), jnp.int32),
        pltpu.VMEM((2, NB, K, 8), F32),
        pltpu.SemaphoreType.DMA((2,)),
        pltpu.SemaphoreType.DMA((2,)),
        pltpu.SemaphoreType.DMA((2,)),
    ],
)(_agg1_body)


def _agg2_body(row1d, col1d, t0, t1, t2, t3, zeros16,
               out0, out1, out2, out3,
               acc, idxr, idxc, rows, semi, semg, sems):
    c = lax.axis_index("c")
    s = lax.axis_index("s")
    base = s * CH2
    tabs = (t0, t1, t2, t3)
    outs = (out0, out1, out2, out3)
    for q in (0, 1):
        pltpu.sync_copy(zeros16, acc.at[pl.ds(s * ZR, ZR)])
        plsc.subcore_barrier()

        @pl.when(c == 0)
        def _():
            _agg_pipeline(row1d, col1d, tabs[q], acc, idxr, idxc, rows,
                          semi, semg, sems, base, CH2 // NB)

        @pl.when(c == 1)
        def _():
            _agg_pipeline(row1d, col1d, tabs[2 + q], acc, idxr, idxc, rows,
                          semi, semg, sems, base, CH2 // NB)

        plsc.subcore_barrier()

        @pl.when(c == 0)
        def _():
            pltpu.sync_copy(acc.at[pl.ds(s * ZR, ZR)],
                            outs[q].at[pl.ds(s * ZR, ZR)])

        @pl.when(c == 1)
        def _():
            pltpu.sync_copy(acc.at[pl.ds(s * ZR, ZR)],
                            outs[2 + q].at[pl.ds(s * ZR, ZR)])
        plsc.subcore_barrier()


_agg2_kernel = functools.partial(
    pl.kernel,
    out_type=tuple(jax.ShapeDtypeStruct((NP, 16), F32) for _ in range(4)),
    mesh=_MESH,
    compiler_params=pltpu.CompilerParams(use_tc_tiling_on_sc=False),
    scratch_types=[
        pltpu.VMEM_SHARED((NP, 16), F32),
        pltpu.VMEM((2, NB, 1, K), jnp.int32),
        pltpu.VMEM((2, NB, 1, K), jnp.int32),
        pltpu.VMEM((2, NB, K, 16), F32),
        pltpu.SemaphoreType.DMA((2,)),
        pltpu.SemaphoreType.DMA((2,)),
        pltpu.SemaphoreType.DMA((2,)),
    ],
)(_agg2_body)


R = 2000
G = N // R


def _tc1_body(degp_ref, x_ref, dinv_ref, u1p_ref):
    deg = degp_ref[0, :, 0:1] + degp_ref[1, :, 0:1] + 1.0
    dinv = lax.rsqrt(deg)
    dinv_ref[...] = dinv
    u1 = x_ref[...] * dinv
    u1p_ref[...] = jnp.concatenate([u1, jnp.zeros((R, 6), F32)], axis=1)


def _tc1(degp, x):
    return pl.pallas_call(
        _tc1_body,
        grid=(G,),
        in_specs=[
            pl.BlockSpec((NC, R, 8), lambda i: (0, i, 0)),
            pl.BlockSpec((R, 2), lambda i: (i, 0)),
        ],
        out_specs=[
            pl.BlockSpec((R, 1), lambda i: (i, 0)),
            pl.BlockSpec((R, 8), lambda i: (i, 0)),
        ],
        out_shape=[
            jax.ShapeDtypeStruct((N, 1), F32),
            jax.ShapeDtypeStruct((N, 8), F32),
        ],
    )(degp, x)


def _tc2_body(aggp_ref, u1p_ref, dinv_ref, w1_ref, b1_ref, w2_ref,
              u2q0_ref, u2q1_ref, u2q2_ref, u2q3_ref):
    agg = (aggp_ref[0] + aggp_ref[1] + u1p_ref[...])[:, 0:2] * dinv_ref[...]
    h1 = jax.nn.relu(agg[:, 0:1] * w1_ref[0:1, :]
                     + agg[:, 1:2] * w1_ref[1:2, :] + b1_ref[...])
    t = jnp.dot(h1, w2_ref[...], preferred_element_type=F32)
    u2 = t * dinv_ref[...]
    u2q0_ref[...] = u2[:, 0:16]
    u2q1_ref[...] = u2[:, 16:32]
    u2q2_ref[...] = u2[:, 32:48]
    u2q3_ref[...] = u2[:, 48:64]


def _tc2(aggp, u1p, dinv, W1, b1, W2):
    return pl.pallas_call(
        _tc2_body,
        grid=(G,),
        in_specs=[
            pl.BlockSpec((NC, R, 8), lambda i: (0, i, 0)),
            pl.BlockSpec((R, 8), lambda i: (i, 0)),
            pl.BlockSpec((R, 1), lambda i: (i, 0)),
            pl.BlockSpec((2, 128), lambda i: (0, 0)),
            pl.BlockSpec((1, 128), lambda i: (0, 0)),
            pl.BlockSpec((128, 64), lambda i: (0, 0)),
        ],
        out_specs=[pl.BlockSpec((R, 16), lambda i: (i, 0))
                   for _ in range(4)],
        out_shape=[jax.ShapeDtypeStruct((N, 16), F32) for _ in range(4)],
    )(aggp, u1p, dinv, W1, b1, W2)


def _tc3_body(a0_ref, a1_ref, a2_ref, a3_ref,
              u0_ref, u1_ref, u2_ref, u3_ref, dinv_ref, b2_ref,
              wa_ref, ba_ref, wp_ref, bp_ref, out_ref):
    dinv = dinv_ref[...]
    arefs = (a0_ref, a1_ref, a2_ref, a3_ref)
    urefs = (u0_ref, u1_ref, u2_ref, u3_ref)
    alog = ba_ref[0, 0]
    plog = 0.0
    for qq in range(4):
        lo = 16 * qq
        h2q = jax.nn.relu((arefs[qq][...] + urefs[qq][...]) * dinv
                          + b2_ref[:, lo:lo + 16])
        alog = alog + jnp.sum(h2q * wa_ref[:, lo:lo + 16], axis=1,
                              keepdims=True)
        plog = plog + jnp.sum(h2q * wp_ref[:, lo:lo + 16], axis=1,
                              keepdims=True)
    attn = jax.nn.sigmoid(alog)
    out_ref[...] = jax.nn.sigmoid(attn * plog + bp_ref[0, 0])


def _tc3(aq, uq, dinv, b2, wa, ba, wp, bp):
    return pl.pallas_call(
        _tc3_body,
        grid=(G,),
        in_specs=[pl.BlockSpec((R, 16), lambda i: (i, 0))
                  for _ in range(8)] + [
            pl.BlockSpec((R, 1), lambda i: (i, 0)),
            pl.BlockSpec((1, 64), lambda i: (0, 0)),
            pl.BlockSpec((1, 64), lambda i: (0, 0)),
            pl.BlockSpec((1, 1), lambda i: (0, 0)),
            pl.BlockSpec((1, 64), lambda i: (0, 0)),
            pl.BlockSpec((1, 1), lambda i: (0, 0)),
        ],
        out_specs=pl.BlockSpec((R, 1), lambda i: (i, 0)),
        out_shape=jax.ShapeDtypeStruct((N, 1), F32),
    )(*aq, *uq, dinv, b2, wa, ba, wp, bp)


def kernel(x, edge_index, W1, b1, W2, b2, Wa, ba, Wp, bp):
    pad = EP - E
    pad_rows = (jnp.arange(pad, dtype=jnp.int32) % N)
    pad_cols = N + (jnp.arange(pad, dtype=jnp.int32) % (NP - N))
    row1d = jnp.concatenate([edge_index[0], pad_rows]).reshape(EP // K, 1, K)
    col1d = jnp.concatenate([edge_index[1], pad_cols]).reshape(EP // K, 1, K)
    ones8 = jnp.ones((K, 8), F32)
    zeros8 = jnp.zeros((ZR, 8), F32)
    zeros16 = jnp.zeros((ZR, 16), F32)

    degp = _deg_kernel(col1d, ones8, zeros8)
    dinv, u1p = _tc1(degp, x)
    aggp = _agg1_kernel(row1d, col1d, u1p, zeros8)
    u2q = _tc2(aggp, u1p, dinv, W1, b1.reshape(1, 128), W2)
    a2q = _agg2_kernel(row1d, col1d, *u2q, zeros16)
    out = _tc3([a[:N] for a in a2q], u2q, dinv, b2.reshape(1, 64),
               Wa.reshape(1, 64), ba.reshape(1, 1),
               Wp.reshape(1, 64), bp.reshape(1, 1))
    return out.reshape(N)


# speedup vs baseline: 42.2424x; 1.1125x over previous
import functools

import jax
import jax.numpy as jnp
from jax import lax
from jax.experimental import pallas as pl
from jax.experimental.pallas import tpu as pltpu
from jax.experimental.pallas import tpu_sc as plsc

N = 50000
E = 1600000
K = 128
NB_A = 8
NB_B = 10
NB_C = 16
NC = 2
NS = 16
NP = 50048
ZR = NP // NS
CH = 400
EP = NC * NS * CH * K
CH2 = 2 * CH

_MESH = plsc.VectorSubcoreMesh(
    core_axis_name="c", subcore_axis_name="s", num_cores=NC, num_subcores=NS)

F32 = jnp.float32


def _fire_idx(src3, dst, pp, o, base_row, sem, nb):
    pltpu.async_copy(src3.at[pl.ds(base_row + o * nb, nb)], dst.at[pp],
                     sem.at[pp])


def _drain_idx(src3, dst, pp, sem, n, nb):
    for _ in range(n):
        pltpu.make_async_copy(src3.at[pl.ds(0, nb)], dst.at[pp],
                              sem.at[pp]).wait()


def _agg_pipeline(src_r, src_c, table, acc, idxr, idxc, rows,
                  semi, semg, sems, base, nblocks, nb):
    def fire_scatters(pp):
        for b in range(nb):
            pltpu.async_copy(rows.at[pp, pl.ds(b * K, K)],
                             acc.at[idxc.at[pp, b, 0]],
                             sems.at[pp], add=True)

    def drain_scatters(pp):
        pltpu.make_async_copy(rows.at[pp], acc.at[pl.ds(0, nb * K)],
                              sems.at[pp]).wait()

    zero = jnp.int32(0)
    _fire_idx(src_r, idxr, 0, zero, base, semi, nb)
    _fire_idx(src_c, idxc, 0, zero, base, semi, nb)
    _fire_idx(src_r, idxr, 1, zero + 1, base, semi, nb)

    def body(oo, _):
        for p in (0, 1):
            q = 1 - p
            o = 2 * oo + p
            _drain_idx(src_r, idxr, p, semi, 2, nb)
            for b in range(nb):
                pltpu.async_copy(table.at[idxr.at[p, b, 0]],
                                 rows.at[p, pl.ds(b * K, K)], semg.at[p])
            pltpu.make_async_copy(table.at[pl.ds(0, nb * K)], rows.at[p],
                                  semg.at[p]).wait()
            _fire_idx(src_r, idxr, p, jnp.minimum(o + 2, nblocks - 1),
                      base, semi, nb)
            if p == 0:
                @pl.when(oo > 0)
                def _():
                    drain_scatters(q)
            else:
                drain_scatters(q)
            _fire_idx(src_c, idxc, q, jnp.minimum(o + 1, nblocks - 1),
                      base, semi, nb)
            fire_scatters(p)
        return 0

    lax.fori_loop(0, nblocks // 2, body, 0)
    _drain_idx(src_r, idxr, 0, semi, 2, nb)
    _drain_idx(src_r, idxr, 1, semi, 1, nb)
    drain_scatters(1)


def _deg_body(col1d, ones_hbm, zeros8, out, acc, idxc, ones_v, semi, sems):
    c = lax.axis_index("c")
    s = lax.axis_index("s")
    pltpu.sync_copy(zeros8, acc.at[pl.ds(s * ZR, ZR)])
    pltpu.sync_copy(ones_hbm, ones_v)
    plsc.subcore_barrier()
    wid = c * NS + s
    base = wid * CH
    nblocks = CH // NB_A

    def fire_scatters(pp):
        for b in range(NB_A):
            pltpu.async_copy(ones_v, acc.at[idxc.at[pp, b, 0]],
                             sems.at[pp], add=True)

    def drain_scatters(pp):
        for _ in range(NB_A):
            pltpu.make_async_copy(ones_v, acc.at[pl.ds(0, K)],
                                  sems.at[pp]).wait()

    _fire_idx(col1d, idxc, 0, jnp.int32(0), base, semi, NB_A)

    def body(oo, _):
        for p in (0, 1):
            q = 1 - p
            o = 2 * oo + p
            _drain_idx(col1d, idxc, p, semi, 1, NB_A)
            if p == 0:
                @pl.when(oo > 0)
                def _():
                    drain_scatters(q)
            else:
                drain_scatters(q)
            _fire_idx(col1d, idxc, q, jnp.minimum(o + 1, nblocks - 1),
                      base, semi, NB_A)
            fire_scatters(p)
        return 0

    lax.fori_loop(0, nblocks // 2, body, 0)
    _drain_idx(col1d, idxc, 0, semi, 1, NB_A)
    drain_scatters(1)

    plsc.subcore_barrier()
    pltpu.sync_copy(acc.at[pl.ds(s * ZR, ZR)], out.at[c, pl.ds(s * ZR, ZR)])


_deg_kernel = functools.partial(
    pl.kernel,
    out_type=jax.ShapeDtypeStruct((NC, NP, 8), F32),
    mesh=_MESH,
    compiler_params=pltpu.CompilerParams(use_tc_tiling_on_sc=False),
    scratch_types=[
        pltpu.VMEM_SHARED((NP, 8), F32),
        pltpu.VMEM((2, NB_A, 1, K), jnp.int32),
        pltpu.VMEM((K, 8), F32),
        pltpu.SemaphoreType.DMA((2,)),
        pltpu.SemaphoreType.DMA((2,)),
    ],
)(_deg_body)


def _agg1_body(row1d, col1d, table, zeros8, out,
               acc, idxr, idxc, rows, semi, semg, sems):
    c = lax.axis_index("c")
    s = lax.axis_index("s")
    pltpu.sync_copy(zeros8, acc.at[pl.ds(s * ZR, ZR)])
    plsc.subcore_barrier()
    wid = c * NS + s
    base = wid * CH
    _agg_pipeline(row1d, col1d, table, acc, idxr, idxc, rows,
                  semi, semg, sems, base, CH // NB_B, NB_B)
    plsc.subcore_barrier()
    pltpu.sync_copy(acc.at[pl.ds(s * ZR, ZR)], out.at[c, pl.ds(s * ZR, ZR)])


_agg1_kernel = functools.partial(
    pl.kernel,
    out_type=jax.ShapeDtypeStruct((NC, NP, 8), F32),
    mesh=_MESH,
    compiler_params=pltpu.CompilerParams(use_tc_tiling_on_sc=False),
    scratch_types=[
        pltpu.VMEM_SHARED((NP, 8), F32),
        pltpu.VMEM((2, NB_B, 1, K), jnp.int32),
        pltpu.VMEM((2, NB_B, 1, K), jnp.int32),
        pltpu.VMEM((2, NB_B * K, 8), F32),
        pltpu.SemaphoreType.DMA((2,)),
        pltpu.SemaphoreType.DMA((2,)),
        pltpu.SemaphoreType.DMA((2,)),
    ],
)(_agg1_body)


def _agg2_body(row1d, col1d, t0, t1, t2, t3, zeros16,
               out0, out1, out2, out3,
               acc, idxr, idxc, rows, semi, semg, sems):
    c = lax.axis_index("c")
    s = lax.axis_index("s")
    base = s * CH2
    tabs = (t0, t1, t2, t3)
    outs = (out0, out1, out2, out3)
    for q in (0, 1):
        pltpu.sync_copy(zeros16, acc.at[pl.ds(s * ZR, ZR)])
        plsc.subcore_barrier()

        @pl.when(c == 0)
        def _():
            _agg_pipeline(row1d, col1d, tabs[q], acc, idxr, idxc, rows,
                          semi, semg, sems, base, CH2 // NB_C, NB_C)

        @pl.when(c == 1)
        def _():
            _agg_pipeline(row1d, col1d, tabs[2 + q], acc, idxr, idxc, rows,
                          semi, semg, sems, base, CH2 // NB_C, NB_C)

        plsc.subcore_barrier()

        @pl.when(c == 0)
        def _():
            pltpu.sync_copy(acc.at[pl.ds(s * ZR, ZR)],
                            outs[q].at[pl.ds(s * ZR, ZR)])

        @pl.when(c == 1)
        def _():
            pltpu.sync_copy(acc.at[pl.ds(s * ZR, ZR)],
                            outs[2 + q].at[pl.ds(s * ZR, ZR)])
        plsc.subcore_barrier()


_agg2_kernel = functools.partial(
    pl.kernel,
    out_type=tuple(jax.ShapeDtypeStruct((NP, 16), F32) for _ in range(4)),
    mesh=_MESH,
    compiler_params=pltpu.CompilerParams(use_tc_tiling_on_sc=False),
    scratch_types=[
        pltpu.VMEM_SHARED((NP, 16), F32),
        pltpu.VMEM((2, NB_C, 1, K), jnp.int32),
        pltpu.VMEM((2, NB_C, 1, K), jnp.int32),
        pltpu.VMEM((2, NB_C * K, 16), F32),
        pltpu.SemaphoreType.DMA((2,)),
        pltpu.SemaphoreType.DMA((2,)),
        pltpu.SemaphoreType.DMA((2,)),
    ],
)(_agg2_body)


R = 2000
G = N // R


def _tc1_body(degp_ref, x_ref, dinv_ref, u1p_ref):
    deg = degp_ref[0, :, 0:1] + degp_ref[1, :, 0:1] + 1.0
    dinv = lax.rsqrt(deg)
    dinv_ref[...] = dinv
    u1 = x_ref[...] * dinv
    u1p_ref[...] = jnp.concatenate([u1, jnp.zeros((R, 6), F32)], axis=1)


def _tc1(degp, x):
    return pl.pallas_call(
        _tc1_body,
        grid=(G,),
        in_specs=[
            pl.BlockSpec((NC, R, 8), lambda i: (0, i, 0)),
            pl.BlockSpec((R, 2), lambda i: (i, 0)),
        ],
        out_specs=[
            pl.BlockSpec((R, 1), lambda i: (i, 0)),
            pl.BlockSpec((R, 8), lambda i: (i, 0)),
        ],
        out_shape=[
            jax.ShapeDtypeStruct((N, 1), F32),
            jax.ShapeDtypeStruct((N, 8), F32),
        ],
    )(degp, x)


def _tc2_body(aggp_ref, u1p_ref, dinv_ref, w1_ref, b1_ref, w2_ref,
              u2q0_ref, u2q1_ref, u2q2_ref, u2q3_ref):
    agg = (aggp_ref[0] + aggp_ref[1] + u1p_ref[...])[:, 0:2] * dinv_ref[...]
    h1 = jax.nn.relu(agg[:, 0:1] * w1_ref[0:1, :]
                     + agg[:, 1:2] * w1_ref[1:2, :] + b1_ref[...])
    t = jnp.dot(h1, w2_ref[...], preferred_element_type=F32)
    u2 = t * dinv_ref[...]
    u2q0_ref[...] = u2[:, 0:16]
    u2q1_ref[...] = u2[:, 16:32]
    u2q2_ref[...] = u2[:, 32:48]
    u2q3_ref[...] = u2[:, 48:64]


def _tc2(aggp, u1p, dinv, W1, b1, W2):
    return pl.pallas_call(
        _tc2_body,
        grid=(G,),
        in_specs=[
            pl.BlockSpec((NC, R, 8), lambda i: (0, i, 0)),
            pl.BlockSpec((R, 8), lambda i: (i, 0)),
            pl.BlockSpec((R, 1), lambda i: (i, 0)),
            pl.BlockSpec((2, 128), lambda i: (0, 0)),
            pl.BlockSpec((1, 128), lambda i: (0, 0)),
            pl.BlockSpec((128, 64), lambda i: (0, 0)),
        ],
        out_specs=[pl.BlockSpec((R, 16), lambda i: (i, 0))
                   for _ in range(4)],
        out_shape=[jax.ShapeDtypeStruct((N, 16), F32) for _ in range(4)],
    )(aggp, u1p, dinv, W1, b1, W2)


def _tc3_body(a0_ref, a1_ref, a2_ref, a3_ref,
              u0_ref, u1_ref, u2_ref, u3_ref, dinv_ref, b2_ref,
              wa_ref, ba_ref, wp_ref, bp_ref, out_ref):
    dinv = dinv_ref[...]
    arefs = (a0_ref, a1_ref, a2_ref, a3_ref)
    urefs = (u0_ref, u1_ref, u2_ref, u3_ref)
    alog = ba_ref[0, 0]
    plog = 0.0
    for qq in range(4):
        lo = 16 * qq
        h2q = jax.nn.relu((arefs[qq][...] + urefs[qq][...]) * dinv
                          + b2_ref[:, lo:lo + 16])
        alog = alog + jnp.sum(h2q * wa_ref[:, lo:lo + 16], axis=1,
                              keepdims=True)
        plog = plog + jnp.sum(h2q * wp_ref[:, lo:lo + 16], axis=1,
                              keepdims=True)
    attn = jax.nn.sigmoid(alog)
    out_ref[...] = jax.nn.sigmoid(attn * plog + bp_ref[0, 0])


def _tc3(aq, uq, dinv, b2, wa, ba, wp, bp):
    return pl.pallas_call(
        _tc3_body,
        grid=(G,),
        in_specs=[pl.BlockSpec((R, 16), lambda i: (i, 0))
                  for _ in range(8)] + [
            pl.BlockSpec((R, 1), lambda i: (i, 0)),
            pl.BlockSpec((1, 64), lambda i: (0, 0)),
            pl.BlockSpec((1, 64), lambda i: (0, 0)),
            pl.BlockSpec((1, 1), lambda i: (0, 0)),
            pl.BlockSpec((1, 64), lambda i: (0, 0)),
            pl.BlockSpec((1, 1), lambda i: (0, 0)),
        ],
        out_specs=pl.BlockSpec((R, 1), lambda i: (i, 0)),
        out_shape=jax.ShapeDtypeStruct((N, 1), F32),
    )(*aq, *uq, dinv, b2, wa, ba, wp, bp)


def kernel(x, edge_index, W1, b1, W2, b2, Wa, ba, Wp, bp):
    pad = EP - E
    pad_rows = (jnp.arange(pad, dtype=jnp.int32) % N)
    pad_cols = N + (jnp.arange(pad, dtype=jnp.int32) % (NP - N))
    row1d = jnp.concatenate([edge_index[0], pad_rows]).reshape(EP // K, 1, K)
    col1d = jnp.concatenate([edge_index[1], pad_cols]).reshape(EP // K, 1, K)
    ones8 = jnp.ones((K, 8), F32)
    zeros8 = jnp.zeros((ZR, 8), F32)
    zeros16 = jnp.zeros((ZR, 16), F32)

    degp = _deg_kernel(col1d, ones8, zeros8)
    dinv, u1p = _tc1(degp, x)
    aggp = _agg1_kernel(row1d, col1d, u1p, zeros8)
    u2q = _tc2(aggp, u1p, dinv, W1, b1.reshape(1, 128), W2)
    a2q = _agg2_kernel(row1d, col1d, *u2q, zeros16)
    out = _tc3([a[:N] for a in a2q], u2q, dinv, b2.reshape(1, 64),
               Wa.reshape(1, 64), ba.reshape(1, 1),
               Wp.reshape(1, 64), bp.reshape(1, 1))
    return out.reshape(N)


# speedup vs baseline: 47.6538x; 1.1281x over previous
import functools

import jax
import jax.numpy as jnp
from jax import lax
from jax.experimental import pallas as pl
from jax.experimental.pallas import tpu as pltpu
from jax.experimental.pallas import tpu_sc as plsc

N = 50000
E = 1600000
K = 128
NB_A = 8
NB_B = 10
NB_C = 16
NC = 2
NS = 16
NP = 50048
ZR = NP // NS
CH = 400
EP = NC * NS * CH * K
CH2 = 2 * CH

_MESH = plsc.VectorSubcoreMesh(
    core_axis_name="c", subcore_axis_name="s", num_cores=NC, num_subcores=NS)

F32 = jnp.float32


def _fire_idx(src3, dst, pp, o, base_row, sem, nb):
    pltpu.async_copy(src3.at[pl.ds(base_row + o * nb, nb)], dst.at[pp],
                     sem.at[pp])


def _drain_idx(src3, dst, pp, sem, n, nb):
    for _ in range(n):
        pltpu.make_async_copy(src3.at[pl.ds(0, nb)], dst.at[pp],
                              sem.at[pp]).wait()


def _agg_pipeline(src_r, src_c, table, acc, idxr, idxc, rows,
                  semi, semg, sems, base, nblocks, nb):
    def fire_scatters(pp):
        for b in range(nb):
            pltpu.async_copy(rows.at[pp, pl.ds(b * K, K)],
                             acc.at[idxc.at[pp, b, 0]],
                             sems.at[pp], add=True)

    def drain_scatters(pp):
        pltpu.make_async_copy(rows.at[pp], acc.at[pl.ds(0, nb * K)],
                              sems.at[pp]).wait()

    zero = jnp.int32(0)
    _fire_idx(src_r, idxr, 0, zero, base, semi, nb)
    _fire_idx(src_c, idxc, 0, zero, base, semi, nb)
    _fire_idx(src_r, idxr, 1, zero + 1, base, semi, nb)

    def body(oo, _):
        for p in (0, 1):
            q = 1 - p
            o = 2 * oo + p
            _drain_idx(src_r, idxr, p, semi, 2, nb)
            for b in range(nb):
                pltpu.async_copy(table.at[idxr.at[p, b, 0]],
                                 rows.at[p, pl.ds(b * K, K)], semg.at[p])
            pltpu.make_async_copy(table.at[pl.ds(0, nb * K)], rows.at[p],
                                  semg.at[p]).wait()
            _fire_idx(src_r, idxr, p, jnp.minimum(o + 2, nblocks - 1),
                      base, semi, nb)
            if p == 0:
                @pl.when(oo > 0)
                def _():
                    drain_scatters(q)
            else:
                drain_scatters(q)
            _fire_idx(src_c, idxc, q, jnp.minimum(o + 1, nblocks - 1),
                      base, semi, nb)
            fire_scatters(p)
        return 0

    lax.fori_loop(0, nblocks // 2, body, 0)
    _drain_idx(src_r, idxr, 0, semi, 2, nb)
    _drain_idx(src_r, idxr, 1, semi, 1, nb)
    drain_scatters(1)


def _deg_body(col1d, ones_hbm, zeros8, out, acc, idxc, ones_v, semi, sems):
    c = lax.axis_index("c")
    s = lax.axis_index("s")
    pltpu.sync_copy(zeros8, acc.at[pl.ds(s * ZR, ZR)])
    pltpu.sync_copy(ones_hbm, ones_v)
    plsc.subcore_barrier()
    wid = c * NS + s
    base = wid * CH
    nblocks = CH // NB_A

    def fire_scatters(pp):
        for b in range(NB_A):
            pltpu.async_copy(ones_v, acc.at[idxc.at[pp, b, 0]],
                             sems.at[pp], add=True)

    def drain_scatters(pp):
        for _ in range(NB_A):
            pltpu.make_async_copy(ones_v, acc.at[pl.ds(0, K)],
                                  sems.at[pp]).wait()

    _fire_idx(col1d, idxc, 0, jnp.int32(0), base, semi, NB_A)

    def body(oo, _):
        for p in (0, 1):
            q = 1 - p
            o = 2 * oo + p
            _drain_idx(col1d, idxc, p, semi, 1, NB_A)
            if p == 0:
                @pl.when(oo > 0)
                def _():
                    drain_scatters(q)
            else:
                drain_scatters(q)
            _fire_idx(col1d, idxc, q, jnp.minimum(o + 1, nblocks - 1),
                      base, semi, NB_A)
            fire_scatters(p)
        return 0

    lax.fori_loop(0, nblocks // 2, body, 0)
    _drain_idx(col1d, idxc, 0, semi, 1, NB_A)
    drain_scatters(1)

    plsc.subcore_barrier()
    pltpu.sync_copy(acc.at[pl.ds(s * ZR, ZR)], out.at[c, pl.ds(s * ZR, ZR)])


_deg_kernel = functools.partial(
    pl.kernel,
    out_type=jax.ShapeDtypeStruct((NC, NP, 8), F32),
    mesh=_MESH,
    compiler_params=pltpu.CompilerParams(use_tc_tiling_on_sc=False),
    scratch_types=[
        pltpu.VMEM_SHARED((NP, 8), F32),
        pltpu.VMEM((2, NB_A, 1, K), jnp.int32),
        pltpu.VMEM((K, 8), F32),
        pltpu.SemaphoreType.DMA((2,)),
        pltpu.SemaphoreType.DMA((2,)),
    ],
)(_deg_body)


def _agg1_body(row1d, col1d, table, zeros8, out,
               acc, idxr, idxc, rows, semi, semg, sems):
    c = lax.axis_index("c")
    s = lax.axis_index("s")
    pltpu.sync_copy(zeros8, acc.at[pl.ds(s * ZR, ZR)])
    plsc.subcore_barrier()
    wid = c * NS + s
    base = wid * CH
    _agg_pipeline(row1d, col1d, table, acc, idxr, idxc, rows,
                  semi, semg, sems, base, CH // NB_B, NB_B)
    plsc.subcore_barrier()
    pltpu.sync_copy(acc.at[pl.ds(s * ZR, ZR)], out.at[c, pl.ds(s * ZR, ZR)])


_agg1_kernel = functools.partial(
    pl.kernel,
    out_type=jax.ShapeDtypeStruct((NC, NP, 8), F32),
    mesh=_MESH,
    compiler_params=pltpu.CompilerParams(use_tc_tiling_on_sc=False),
    scratch_types=[
        pltpu.VMEM_SHARED((NP, 8), F32),
        pltpu.VMEM((2, NB_B, 1, K), jnp.int32),
        pltpu.VMEM((2, NB_B, 1, K), jnp.int32),
        pltpu.VMEM((2, NB_B * K, 8), F32),
        pltpu.SemaphoreType.DMA((2,)),
        pltpu.SemaphoreType.DMA((2,)),
        pltpu.SemaphoreType.DMA((2,)),
    ],
)(_agg1_body)


def _agg2_body(row1d, col1d, t0, t1, t2, t3, zeros16,
               outw,
               acc, idxr, idxc, rows, semi, semg, sems):
    c = lax.axis_index("c")
    s = lax.axis_index("s")
    base = s * CH2
    tabs = (t0, t1, t2, t3)
    for q in (0, 1):
        pltpu.sync_copy(zeros16, acc.at[pl.ds(s * ZR, ZR)])
        plsc.subcore_barrier()

        @pl.when(c == 0)
        def _():
            _agg_pipeline(row1d, col1d, tabs[q], acc, idxr, idxc, rows,
                          semi, semg, sems, base, CH2 // NB_C, NB_C)

        @pl.when(c == 1)
        def _():
            _agg_pipeline(row1d, col1d, tabs[2 + q], acc, idxr, idxc, rows,
                          semi, semg, sems, base, CH2 // NB_C, NB_C)

        plsc.subcore_barrier()
        pltpu.sync_copy(acc.at[pl.ds(s * ZR, ZR)],
                        outw.at[pl.ds(s * ZR, ZR), pl.ds(32 * c + 16 * q, 16)])
        plsc.subcore_barrier()


_agg2_kernel = functools.partial(
    pl.kernel,
    out_type=jax.ShapeDtypeStruct((NP, 64), F32),
    mesh=_MESH,
    compiler_params=pltpu.CompilerParams(use_tc_tiling_on_sc=False),
    scratch_types=[
        pltpu.VMEM_SHARED((NP, 16), F32),
        pltpu.VMEM((2, NB_C, 1, K), jnp.int32),
        pltpu.VMEM((2, NB_C, 1, K), jnp.int32),
        pltpu.VMEM((2, NB_C * K, 16), F32),
        pltpu.SemaphoreType.DMA((2,)),
        pltpu.SemaphoreType.DMA((2,)),
        pltpu.SemaphoreType.DMA((2,)),
    ],
)(_agg2_body)


R = 2000
G = N // R


def _tc1_body(degp_ref, x_ref, dinv_ref, u1p_ref):
    deg = degp_ref[0, :, 0:1] + degp_ref[1, :, 0:1] + 1.0
    dinv = lax.rsqrt(deg)
    dinv_ref[...] = dinv
    u1 = x_ref[...] * dinv
    u1p_ref[...] = jnp.concatenate([u1, jnp.zeros((R, 6), F32)], axis=1)


def _tc1(degp, x):
    return pl.pallas_call(
        _tc1_body,
        grid=(G,),
        in_specs=[
            pl.BlockSpec((NC, R, 8), lambda i: (0, i, 0)),
            pl.BlockSpec((R, 2), lambda i: (i, 0)),
        ],
        out_specs=[
            pl.BlockSpec((R, 1), lambda i: (i, 0)),
            pl.BlockSpec((R, 8), lambda i: (i, 0)),
        ],
        out_shape=[
            jax.ShapeDtypeStruct((N, 1), F32),
            jax.ShapeDtypeStruct((N, 8), F32),
        ],
    )(degp, x)


def _tc2_body(aggp_ref, u1p_ref, dinv_ref, w1_ref, b1_ref, w2_ref,
              u2q0_ref, u2q1_ref, u2q2_ref, u2q3_ref, u2w_ref):
    agg = (aggp_ref[0] + aggp_ref[1] + u1p_ref[...])[:, 0:2] * dinv_ref[...]
    h1 = jax.nn.relu(agg[:, 0:1] * w1_ref[0:1, :]
                     + agg[:, 1:2] * w1_ref[1:2, :] + b1_ref[...])
    t = jnp.dot(h1, w2_ref[...], preferred_element_type=F32)
    u2 = t * dinv_ref[...]
    u2q0_ref[...] = u2[:, 0:16]
    u2q1_ref[...] = u2[:, 16:32]
    u2q2_ref[...] = u2[:, 32:48]
    u2q3_ref[...] = u2[:, 48:64]
    u2w_ref[...] = u2


def _tc2(aggp, u1p, dinv, W1, b1, W2):
    return pl.pallas_call(
        _tc2_body,
        grid=(G,),
        in_specs=[
            pl.BlockSpec((NC, R, 8), lambda i: (0, i, 0)),
            pl.BlockSpec((R, 8), lambda i: (i, 0)),
            pl.BlockSpec((R, 1), lambda i: (i, 0)),
            pl.BlockSpec((2, 128), lambda i: (0, 0)),
            pl.BlockSpec((1, 128), lambda i: (0, 0)),
            pl.BlockSpec((128, 64), lambda i: (0, 0)),
        ],
        out_specs=[pl.BlockSpec((R, 16), lambda i: (i, 0))
                   for _ in range(4)] + [pl.BlockSpec((R, 64),
                                                      lambda i: (i, 0))],
        out_shape=[jax.ShapeDtypeStruct((N, 16), F32) for _ in range(4)]
        + [jax.ShapeDtypeStruct((N, 64), F32)],
    )(aggp, u1p, dinv, W1, b1, W2)


def _tc3_body(aw_ref, uw_ref, dinv_ref, b2_ref,
              wa_ref, ba_ref, wp_ref, bp_ref, out_ref):
    dinv = dinv_ref[...]
    h2 = jax.nn.relu((aw_ref[...] + uw_ref[...]) * dinv + b2_ref[...])
    alog = jnp.sum(h2 * wa_ref[...], axis=1, keepdims=True) + ba_ref[0, 0]
    attn = jax.nn.sigmoid(alog)
    plog = jnp.sum(h2 * wp_ref[...], axis=1, keepdims=True)
    out_ref[...] = jax.nn.sigmoid(attn * plog + bp_ref[0, 0])


def _tc3(aw, uw, dinv, b2, wa, ba, wp, bp):
    return pl.pallas_call(
        _tc3_body,
        grid=(G,),
        in_specs=[
            pl.BlockSpec((R, 64), lambda i: (i, 0)),
            pl.BlockSpec((R, 64), lambda i: (i, 0)),
            pl.BlockSpec((R, 1), lambda i: (i, 0)),
            pl.BlockSpec((1, 64), lambda i: (0, 0)),
            pl.BlockSpec((1, 64), lambda i: (0, 0)),
            pl.BlockSpec((1, 1), lambda i: (0, 0)),
            pl.BlockSpec((1, 64), lambda i: (0, 0)),
            pl.BlockSpec((1, 1), lambda i: (0, 0)),
        ],
        out_specs=pl.BlockSpec((R, 1), lambda i: (i, 0)),
        out_shape=jax.ShapeDtypeStruct((N, 1), F32),
    )(aw, uw, dinv, b2, wa, ba, wp, bp)


def kernel(x, edge_index, W1, b1, W2, b2, Wa, ba, Wp, bp):
    pad = EP - E
    pad_rows = (jnp.arange(pad, dtype=jnp.int32) % N)
    pad_cols = N + (jnp.arange(pad, dtype=jnp.int32) % (NP - N))
    row1d = jnp.concatenate([edge_index[0], pad_rows]).reshape(EP // K, 1, K)
    col1d = jnp.concatenate([edge_index[1], pad_cols]).reshape(EP // K, 1, K)
    ones8 = jnp.ones((K, 8), F32)
    zeros8 = jnp.zeros((ZR, 8), F32)
    zeros16 = jnp.zeros((ZR, 16), F32)

    degp = _deg_kernel(col1d, ones8, zeros8)
    dinv, u1p = _tc1(degp, x)
    aggp = _agg1_kernel(row1d, col1d, u1p, zeros8)
    q0, q1, q2, q3, u2w = _tc2(aggp, u1p, dinv, W1, b1.reshape(1, 128), W2)
    a2w = _agg2_kernel(row1d, col1d, q0, q1, q2, q3, zeros16)
    out = _tc3(a2w, u2w, dinv, b2.reshape(1, 64),
               Wa.reshape(1, 64), ba.reshape(1, 1),
               Wp.reshape(1, 64), bp.reshape(1, 1))
    return out.reshape(N)
